# baseline (speedup 1.0000x reference)
import functools

import jax
import jax.numpy as jnp
from jax import lax
from jax.experimental import pallas as pl
from jax.experimental.pallas import tpu as pltpu
from jax.experimental.pallas import tpu_sc as plsc

_N = 10000
_E = 320000
_D = 128
_HD = 64
_EA = 128

_NC = 2
_NS = 16
_NW = _NC * _NS
_CH = 128
_NCH = 80
_EPW = _CH * _NCH
_EPAD = _NW * _EPW
_TR = 16
_RPS = 624
_RLAST = _N - (_NS - 1) * _RPS


def _per_subcore_slices(s, fn):
    @pl.when(s < _NS - 1)
    def _():
        fn(pl.multiple_of(s * _RPS, 8), _RPS)

    @pl.when(s == _NS - 1)
    def _():
        fn((_NS - 1) * _RPS, _RLAST)


def _per_core_out(c, src_fn, out_a, out_b):
    @pl.when(c == 0)
    def _():
        src_fn(out_a)

    @pl.when(c == 1)
    def _():
        src_fn(out_b)


def _sc_spmm_body(h_hbm, ridx3_hbm, colp_hbm, z128,
                  sa_out, sb_out,
                  ridx_v, cidx_a, cidx_b, rows_a, rows_b,
                  gsem_a, gsem_b, csem_a, csem_b, ssem_a, ssem_b, s_sh):
    c = lax.axis_index("c")
    s = lax.axis_index("s")
    _per_subcore_slices(
        s, lambda off, sz: pltpu.sync_copy(z128.at[pl.ds(off, sz)],
                                           s_sh.at[pl.ds(off, sz)]))
    w = c * _NS + s
    pltpu.sync_copy(ridx3_hbm.at[w], ridx_v)
    plsc.subcore_barrier()

    base_w = w * _EPW

    def pair(j, carry):
        a = 2 * j
        da = pltpu.async_copy(h_hbm.at[ridx_v.at[a]], rows_a, gsem_a)
        dca = pltpu.async_copy(
            colp_hbm.at[pl.ds(base_w + a * _CH, _CH)], cidx_a, csem_a)
        db = pltpu.async_copy(h_hbm.at[ridx_v.at[a + 1]], rows_b, gsem_b)
        dcb = pltpu.async_copy(
            colp_hbm.at[pl.ds(base_w + (a + 1) * _CH, _CH)], cidx_b, csem_b)
        da.wait()
        dca.wait()
        pltpu.sync_copy(rows_a, s_sh.at[cidx_a], add=True)
        db.wait()
        dcb.wait()
        pltpu.sync_copy(rows_b, s_sh.at[cidx_b], add=True)
        return carry

    lax.fori_loop(0, _NCH // 2, pair, 0)
    plsc.subcore_barrier()

    def _out(off, sz):
        _per_core_out(
            c, lambda o: pltpu.sync_copy(s_sh.at[pl.ds(off, sz)],
                                         o.at[pl.ds(off, sz)]),
            sa_out, sb_out)

    _per_subcore_slices(s, _out)


def _sc_agg_body(ea_hbm, ridx3_hbm, z128,
                 agga_out, aggb_out,
                 ridx_v, ea_a, ea_b, lsem_a, lsem_b, ssem_a, ssem_b, agg_sh):
    c = lax.axis_index("c")
    s = lax.axis_index("s")
    _per_subcore_slices(
        s, lambda off, sz: pltpu.sync_copy(z128.at[pl.ds(off, sz)],
                                           agg_sh.at[pl.ds(off, sz)]))
    w = c * _NS + s
    pltpu.sync_copy(ridx3_hbm.at[w], ridx_v)
    plsc.subcore_barrier()

    base_w = w * _EPW

    def pair(j, carry):
        a = 2 * j
        da = pltpu.async_copy(
            ea_hbm.at[pl.ds(base_w + a * _CH, _CH)], ea_a, lsem_a)
        db = pltpu.async_copy(
            ea_hbm.at[pl.ds(base_w + (a + 1) * _CH, _CH)], ea_b, lsem_b)
        da.wait()
        pltpu.sync_copy(ea_a, agg_sh.at[ridx_v.at[a]], add=True)
        db.wait()
        pltpu.sync_copy(ea_b, agg_sh.at[ridx_v.at[a + 1]], add=True)
        return carry

    lax.fori_loop(0, _NCH // 2, pair, 0)
    plsc.subcore_barrier()

    def _out(off, sz):
        _per_core_out(
            c, lambda o: pltpu.sync_copy(agg_sh.at[pl.ds(off, sz)],
                                         o.at[pl.ds(off, sz)]),
            agga_out, aggb_out)

    _per_subcore_slices(s, _out)


_f32 = jnp.float32
_nd = jax.ShapeDtypeStruct((_N, _D), _f32)
_nh = jax.ShapeDtypeStruct((_N, _HD), _f32)
_na = jax.ShapeDtypeStruct((_N, _EA), _f32)


@functools.cache
def _build_sc_kernels():
    mesh = plsc.VectorSubcoreMesh(core_axis_name="c", subcore_axis_name="s")
    sc_spmm = pl.kernel(
        _sc_spmm_body,
        out_type=(_nd, _nd),
        mesh=mesh,
        scratch_types=[
            pltpu.VMEM((_NCH, _CH), jnp.int32),
            pltpu.VMEM((_CH,), jnp.int32),
            pltpu.VMEM((_CH,), jnp.int32),
            pltpu.VMEM((_CH, _D), jnp.float32),
            pltpu.VMEM((_CH, _D), jnp.float32),
            pltpu.SemaphoreType.DMA,
            pltpu.SemaphoreType.DMA,
            pltpu.SemaphoreType.DMA,
            pltpu.SemaphoreType.DMA,
            pltpu.SemaphoreType.DMA,
            pltpu.SemaphoreType.DMA,
            pltpu.VMEM_SHARED((_N + _TR, _D), jnp.float32),
        ],
    )
    sc_agg = pl.kernel(
        _sc_agg_body,
        out_type=(_na, _na),
        mesh=mesh,
        scratch_types=[
            pltpu.VMEM((_NCH, _CH), jnp.int32),
            pltpu.VMEM((_CH, _EA), jnp.float32),
            pltpu.VMEM((_CH, _EA), jnp.float32),
            pltpu.SemaphoreType.DMA,
            pltpu.SemaphoreType.DMA,
            pltpu.SemaphoreType.DMA,
            pltpu.SemaphoreType.DMA,
            pltpu.VMEM_SHARED((_N, _EA), jnp.float32),
        ],
    )
    return sc_spmm, sc_agg


def _bn(y, g, b, eps=1e-5):
    m = jnp.mean(y, axis=0, keepdims=True)
    v = jnp.mean((y - m) * (y - m), axis=0, keepdims=True)
    return g * (y - m) * lax.rsqrt(v + eps) + b


def _dot(a, w):
    return jnp.dot(a, w, preferred_element_type=jnp.float32,
                   precision=lax.Precision.HIGHEST)


def _tc_pre_body(x_ref, Wnb_ref, bnb_ref, Wn_ref, bn_ref, gn_ref, btn_ref,
                 h_ref, xn_ref):
    x = x_ref[...]
    h_ref[...] = _dot(x, Wnb_ref[...]) + bnb_ref[...]
    xn_ref[...] = _bn(_dot(x, Wn_ref[...]) + bn_ref[...], gn_ref[...], btn_ref[...])


def _tc_layer0_body(sa_ref, sb_ref, agga_ref, aggb_ref, xn0_ref,
                    Wea_ref, Wm1_ref, bm1_ref, Wm2_ref, bm2_ref,
                    ge_ref, bte_ref, gnb_ref, btnb_ref,
                    gm1_ref, btm1_ref, gm2_ref, btm2_ref,
                    ea_ref, x1_ref):
    msg = sa_ref[...] + sb_ref[...]
    agg = agga_ref[...] + aggb_ref[...]
    ea_agg = _bn(_dot(agg, Wea_ref[...]), ge_ref[...], bte_ref[...])
    ea_ref[...] = ea_agg
    out = jnp.maximum(
        xn0_ref[...] + _bn(msg, gnb_ref[...], btnb_ref[...]) + ea_agg, 0.0)
    out = _bn(_dot(out, Wm1_ref[...]) + bm1_ref[...], gm1_ref[...], btm1_ref[...])
    out = jnp.maximum(out, 0.0)
    out = _bn(_dot(out, Wm2_ref[...]) + bm2_ref[...], gm2_ref[...], btm2_ref[...])
    x1_ref[...] = jnp.maximum(out, 0.0)


def _tc_layer1_body(sa_ref, sb_ref, xn1_ref, ea_ref,
                    Wm1_ref, bm1_ref, Wm2_ref, bm2_ref,
                    gnb_ref, btnb_ref, gm1_ref, btm1_ref, gm2_ref, btm2_ref,
                    x2_ref):
    msg = sa_ref[...] + sb_ref[...]
    out = jnp.maximum(
        xn1_ref[...] + _bn(msg, gnb_ref[...], btnb_ref[...]) + ea_ref[...], 0.0)
    out = _bn(_dot(out, Wm1_ref[...]) + bm1_ref[...], gm1_ref[...], btm1_ref[...])
    out = jnp.maximum(out, 0.0)
    out = _bn(_dot(out, Wm2_ref[...]) + bm2_ref[...], gm2_ref[...], btm2_ref[...])
    x2_ref[...] = jnp.maximum(out, 0.0)


_tc_pre = pl.pallas_call(_tc_pre_body, out_shape=(_nd, _nd))
_tc_layer0 = pl.pallas_call(_tc_layer0_body, out_shape=(_nd, _nd))
_tc_layer1 = pl.pallas_call(_tc_layer1_body, out_shape=_nd)


def kernel(node_attr, edge_index, edge_attr,
           W_node0, b_node0, W_node1, b_node1,
           W_nb0, b_nb0, W_nb1, b_nb1,
           W_edge, b_edge, W_m1, b_m1, W_m2, b_m2,
           g_bn_node, beta_bn_node, g_bn_edge, beta_bn_edge,
           g_bn_nb, beta_bn_nb, g_bn_m1, beta_bn_m1,
           g_bn_m2, beta_bn_m2):
    pad = _EPAD - _E
    rowp = jnp.concatenate([edge_index[0], jnp.arange(pad, dtype=jnp.int32) % _N])
    colp = jnp.concatenate(
        [edge_index[1], _N + (jnp.arange(pad, dtype=jnp.int32) % _TR)])
    row3 = rowp.reshape(_NW, _NCH, _CH)
    ea_aug = jnp.concatenate(
        [jnp.concatenate([edge_attr, jnp.ones((_E, 1), _f32),
                          jnp.zeros((_E, _EA - 17), _f32)], axis=1),
         jnp.zeros((pad, _EA), _f32)], axis=0)
    W_ea = jnp.concatenate(
        [W_edge, b_edge[None, :], jnp.zeros((_EA - 17, _D), _f32)], axis=0)
    z128 = jnp.zeros((_N, _EA), _f32)
    r = lambda v: v[None, :]

    h0, xn0 = _tc_pre(node_attr, W_nb0, r(b_nb0), W_node0, r(b_node0),
                      r(g_bn_node), r(beta_bn_node))

    sc_spmm, sc_agg = _build_sc_kernels()
    agga, aggb = sc_agg(ea_aug, row3, z128)
    z128_dep = z128 + agga[:, :1] * 0.0
    s0a, s0b = sc_spmm(h0, row3, colp, z128_dep)

    ea_agg, x1 = _tc_layer0(
        s0a, s0b, agga, aggb, xn0,
        W_ea, W_m1, r(b_m1), W_m2, r(b_m2),
        r(g_bn_edge), r(beta_bn_edge),
        r(g_bn_nb), r(beta_bn_nb), r(g_bn_m1), r(beta_bn_m1),
        r(g_bn_m2), r(beta_bn_m2))

    h1, xn1 = _tc_pre(x1, W_nb1, r(b_nb1), W_node1, r(b_node1),
                      r(g_bn_node), r(beta_bn_node))

    s1a, s1b = sc_spmm(h1, row3, colp, z128)

    x2 = _tc_layer1(
        s1a, s1b, xn1, ea_agg,
        W_m1, r(b_m1), W_m2, r(b_m2),
        r(g_bn_nb), r(beta_bn_nb), r(g_bn_m1), r(beta_bn_m1),
        r(g_bn_m2), r(beta_bn_m2))
    return x2


# speedup vs baseline: 1.1046x; 1.1046x over previous
import functools

import jax
import jax.numpy as jnp
from jax import lax
from jax.experimental import pallas as pl
from jax.experimental.pallas import tpu as pltpu
from jax.experimental.pallas import tpu_sc as plsc

_N = 10000
_E = 320000
_D = 128
_HD = 64
_EA = 128

_NC = 2
_NS = 16
_NW = _NC * _NS
_CH = 128
_NCH = 80
_EPW = _CH * _NCH
_EPAD = _NW * _EPW
_TR = 16
_RPS = 624
_RLAST = _N - (_NS - 1) * _RPS


def _per_subcore_slices(s, fn):
    @pl.when(s < _NS - 1)
    def _():
        fn(pl.multiple_of(s * _RPS, 8), _RPS)

    @pl.when(s == _NS - 1)
    def _():
        fn((_NS - 1) * _RPS, _RLAST)


def _per_core_out(c, src_fn, out_a, out_b):
    @pl.when(c == 0)
    def _():
        src_fn(out_a)

    @pl.when(c == 1)
    def _():
        src_fn(out_b)


def _sc_spmm_body(h_hbm, ridx3_hbm, colp_hbm, z128,
                  sa_out, sb_out,
                  ridx_v, cidx_a, cidx_b, rows_a, rows_b,
                  gsem_a, gsem_b, csem_a, csem_b, ssem_a, ssem_b, s_sh):
    c = lax.axis_index("c")
    s = lax.axis_index("s")
    _per_subcore_slices(
        s, lambda off, sz: pltpu.sync_copy(z128.at[pl.ds(off, sz)],
                                           s_sh.at[pl.ds(off, sz)]))
    w = c * _NS + s
    pltpu.sync_copy(ridx3_hbm.at[w], ridx_v)
    plsc.subcore_barrier()

    base_w = w * _EPW

    def pair(j, carry):
        a = 2 * j
        da = pltpu.async_copy(h_hbm.at[ridx_v.at[a]], rows_a, gsem_a)
        dca = pltpu.async_copy(
            colp_hbm.at[pl.ds(base_w + a * _CH, _CH)], cidx_a, csem_a)
        db = pltpu.async_copy(h_hbm.at[ridx_v.at[a + 1]], rows_b, gsem_b)
        dcb = pltpu.async_copy(
            colp_hbm.at[pl.ds(base_w + (a + 1) * _CH, _CH)], cidx_b, csem_b)
        da.wait()
        dca.wait()
        pltpu.sync_copy(rows_a, s_sh.at[cidx_a], add=True)
        db.wait()
        dcb.wait()
        pltpu.sync_copy(rows_b, s_sh.at[cidx_b], add=True)
        return carry

    lax.fori_loop(0, _NCH // 2, pair, 0)
    plsc.subcore_barrier()

    def _out(off, sz):
        _per_core_out(
            c, lambda o: pltpu.sync_copy(s_sh.at[pl.ds(off, sz)],
                                         o.at[pl.ds(off, sz)]),
            sa_out, sb_out)

    _per_subcore_slices(s, _out)


def _sc_agg_body(ea_hbm, ridx3_hbm, z128,
                 agga_out, aggb_out,
                 ridx_v, ea_a, ea_b, lsem_a, lsem_b, ssem_a, ssem_b, agg_sh):
    c = lax.axis_index("c")
    s = lax.axis_index("s")
    _per_subcore_slices(
        s, lambda off, sz: pltpu.sync_copy(z128.at[pl.ds(off, sz)],
                                           agg_sh.at[pl.ds(off, sz)]))
    w = c * _NS + s
    pltpu.sync_copy(ridx3_hbm.at[w], ridx_v)
    plsc.subcore_barrier()

    base_w = w * _EPW

    def pair(j, carry):
        a = 2 * j
        da = pltpu.async_copy(
            ea_hbm.at[pl.ds(base_w + a * _CH, _CH)], ea_a, lsem_a)
        db = pltpu.async_copy(
            ea_hbm.at[pl.ds(base_w + (a + 1) * _CH, _CH)], ea_b, lsem_b)
        da.wait()
        pltpu.sync_copy(ea_a, agg_sh.at[ridx_v.at[a]], add=True)
        db.wait()
        pltpu.sync_copy(ea_b, agg_sh.at[ridx_v.at[a + 1]], add=True)
        return carry

    lax.fori_loop(0, _NCH // 2, pair, 0)
    plsc.subcore_barrier()

    def _out(off, sz):
        _per_core_out(
            c, lambda o: pltpu.sync_copy(agg_sh.at[pl.ds(off, sz)],
                                         o.at[pl.ds(off, sz)]),
            agga_out, aggb_out)

    _per_subcore_slices(s, _out)


_f32 = jnp.float32
_nd = jax.ShapeDtypeStruct((_N, _D), _f32)
_nh = jax.ShapeDtypeStruct((_N, _HD), _f32)
_na = jax.ShapeDtypeStruct((_N, _EA), _f32)


@functools.cache
def _build_sc_kernels():
    mesh = plsc.VectorSubcoreMesh(core_axis_name="c", subcore_axis_name="s")
    sc_spmm = pl.kernel(
        _sc_spmm_body,
        out_type=(_nd, _nd),
        mesh=mesh,
        scratch_types=[
            pltpu.VMEM((_NCH, _CH), jnp.int32),
            pltpu.VMEM((_CH,), jnp.int32),
            pltpu.VMEM((_CH,), jnp.int32),
            pltpu.VMEM((_CH, _D), jnp.float32),
            pltpu.VMEM((_CH, _D), jnp.float32),
            pltpu.SemaphoreType.DMA,
            pltpu.SemaphoreType.DMA,
            pltpu.SemaphoreType.DMA,
            pltpu.SemaphoreType.DMA,
            pltpu.SemaphoreType.DMA,
            pltpu.SemaphoreType.DMA,
            pltpu.VMEM_SHARED((_N + _TR, _D), jnp.float32),
        ],
    )
    sc_agg = pl.kernel(
        _sc_agg_body,
        out_type=(_na, _na),
        mesh=mesh,
        scratch_types=[
            pltpu.VMEM((_NCH, _CH), jnp.int32),
            pltpu.VMEM((_CH, _EA), jnp.float32),
            pltpu.VMEM((_CH, _EA), jnp.float32),
            pltpu.SemaphoreType.DMA,
            pltpu.SemaphoreType.DMA,
            pltpu.SemaphoreType.DMA,
            pltpu.SemaphoreType.DMA,
            pltpu.VMEM_SHARED((_N, _EA), jnp.float32),
        ],
    )
    return sc_spmm, sc_agg


def _bn(y, g, b, eps=1e-5):
    m = jnp.mean(y, axis=0, keepdims=True)
    v = jnp.mean((y - m) * (y - m), axis=0, keepdims=True)
    return g * (y - m) * lax.rsqrt(v + eps) + b


def _dot(a, w):
    return jnp.dot(a, w, preferred_element_type=jnp.float32,
                   precision=lax.Precision.HIGHEST)


def _tc_pre_body(x_ref, Wnb_ref, bnb_ref, Wn_ref, bn_ref, gn_ref, btn_ref,
                 h_ref, xn_ref):
    x = x_ref[...]
    h_ref[...] = _dot(x, Wnb_ref[...]) + bnb_ref[...]
    xn_ref[...] = _bn(_dot(x, Wn_ref[...]) + bn_ref[...], gn_ref[...], btn_ref[...])


def _tc_layer0_body(sa_ref, sb_ref, agga_ref, aggb_ref, xn0_ref,
                    Wea_ref, Wm1_ref, bm1_ref, Wm2_ref, bm2_ref,
                    ge_ref, bte_ref, gnb_ref, btnb_ref,
                    gm1_ref, btm1_ref, gm2_ref, btm2_ref,
                    ea_ref, x1_ref):
    msg = sa_ref[...] + sb_ref[...]
    agg = agga_ref[...] + aggb_ref[...]
    ea_agg = _bn(_dot(agg, Wea_ref[...]), ge_ref[...], bte_ref[...])
    ea_ref[...] = ea_agg
    out = jnp.maximum(
        xn0_ref[...] + _bn(msg, gnb_ref[...], btnb_ref[...]) + ea_agg, 0.0)
    out = _bn(_dot(out, Wm1_ref[...]) + bm1_ref[...], gm1_ref[...], btm1_ref[...])
    out = jnp.maximum(out, 0.0)
    out = _bn(_dot(out, Wm2_ref[...]) + bm2_ref[...], gm2_ref[...], btm2_ref[...])
    x1_ref[...] = jnp.maximum(out, 0.0)


def _tc_layer1_body(sa_ref, sb_ref, xn1_ref, ea_ref,
                    Wm1_ref, bm1_ref, Wm2_ref, bm2_ref,
                    gnb_ref, btnb_ref, gm1_ref, btm1_ref, gm2_ref, btm2_ref,
                    x2_ref):
    msg = sa_ref[...] + sb_ref[...]
    out = jnp.maximum(
        xn1_ref[...] + _bn(msg, gnb_ref[...], btnb_ref[...]) + ea_ref[...], 0.0)
    out = _bn(_dot(out, Wm1_ref[...]) + bm1_ref[...], gm1_ref[...], btm1_ref[...])
    out = jnp.maximum(out, 0.0)
    out = _bn(_dot(out, Wm2_ref[...]) + bm2_ref[...], gm2_ref[...], btm2_ref[...])
    x2_ref[...] = jnp.maximum(out, 0.0)


_tc_pre = pl.pallas_call(_tc_pre_body, out_shape=(_nd, _nd))
_tc_layer0 = pl.pallas_call(_tc_layer0_body, out_shape=(_nd, _nd))
_tc_layer1 = pl.pallas_call(_tc_layer1_body, out_shape=_nd)


def kernel(node_attr, edge_index, edge_attr,
           W_node0, b_node0, W_node1, b_node1,
           W_nb0, b_nb0, W_nb1, b_nb1,
           W_edge, b_edge, W_m1, b_m1, W_m2, b_m2,
           g_bn_node, beta_bn_node, g_bn_edge, beta_bn_edge,
           g_bn_nb, beta_bn_nb, g_bn_m1, beta_bn_m1,
           g_bn_m2, beta_bn_m2):
    pad = _EPAD - _E
    rowp = jnp.concatenate([edge_index[0], jnp.arange(pad, dtype=jnp.int32) % _N])
    colp = jnp.concatenate(
        [edge_index[1], _N + (jnp.arange(pad, dtype=jnp.int32) % _TR)])
    row3 = rowp.reshape(_NW, _NCH, _CH)
    ea_aug = jnp.concatenate(
        [jnp.concatenate([edge_attr, jnp.ones((_E, 1), _f32),
                          jnp.zeros((_E, _EA - 17), _f32)], axis=1),
         jnp.zeros((pad, _EA), _f32)], axis=0)
    W_ea = jnp.concatenate(
        [W_edge, b_edge[None, :], jnp.zeros((_EA - 17, _D), _f32)], axis=0)
    z128 = jnp.zeros((_N, _EA), _f32)
    r = lambda v: v[None, :]

    h0, xn0 = _tc_pre(node_attr, W_nb0, r(b_nb0), W_node0, r(b_node0),
                      r(g_bn_node), r(beta_bn_node))

    sc_spmm, sc_agg = _build_sc_kernels()
    s0a, s0b = sc_spmm(h0, row3, colp, z128)
    z128_dep = z128 + s0a[:, :1] * 0.0
    agga, aggb = sc_agg(ea_aug, row3, z128_dep)

    ea_agg, x1 = _tc_layer0(
        s0a, s0b, agga, aggb, xn0,
        W_ea, W_m1, r(b_m1), W_m2, r(b_m2),
        r(g_bn_edge), r(beta_bn_edge),
        r(g_bn_nb), r(beta_bn_nb), r(g_bn_m1), r(beta_bn_m1),
        r(g_bn_m2), r(beta_bn_m2))

    h1, xn1 = _tc_pre(x1, W_nb1, r(b_nb1), W_node1, r(b_node1),
                      r(g_bn_node), r(beta_bn_node))

    s1a, s1b = sc_spmm(h1, row3, colp, z128)

    x2 = _tc_layer1(
        s1a, s1b, xn1, ea_agg,
        W_m1, r(b_m1), W_m2, r(b_m2),
        r(g_bn_nb), r(beta_bn_nb), r(g_bn_m1), r(beta_bn_m1),
        r(g_bn_m2), r(beta_bn_m2))
    return x2


# speedup vs baseline: 1.1789x; 1.0673x over previous
import functools

import jax
import jax.numpy as jnp
from jax import lax
from jax.experimental import pallas as pl
from jax.experimental.pallas import tpu as pltpu
from jax.experimental.pallas import tpu_sc as plsc

_N = 10000
_E = 320000
_D = 128
_HD = 64
_EA = 128

_NC = 2
_NS = 16
_NW = _NC * _NS
_CH = 128
_NCH = 80
_EPW = _CH * _NCH
_EPAD = _NW * _EPW
_TR = 16
_RPS = 624
_RLAST = _N - (_NS - 1) * _RPS


def _per_subcore_slices(s, fn):
    @pl.when(s < _NS - 1)
    def _():
        fn(pl.multiple_of(s * _RPS, 8), _RPS)

    @pl.when(s == _NS - 1)
    def _():
        fn((_NS - 1) * _RPS, _RLAST)


def _per_core_out(c, src_fn, out_a, out_b):
    @pl.when(c == 0)
    def _():
        src_fn(out_a)

    @pl.when(c == 1)
    def _():
        src_fn(out_b)


def _sc_spmm_body(h_hbm, ridx3_hbm, colp_hbm, z128,
                  sa_out, sb_out,
                  ridx_v, cidx_a, cidx_b, rows_a, rows_b,
                  gsem_a, gsem_b, csem_a, csem_b, ssem_a, ssem_b, s_sh):
    c = lax.axis_index("c")
    s = lax.axis_index("s")
    _per_subcore_slices(
        s, lambda off, sz: pltpu.sync_copy(z128.at[pl.ds(off, sz)],
                                           s_sh.at[pl.ds(off, sz)]))
    w = c * _NS + s
    pltpu.sync_copy(ridx3_hbm.at[w], ridx_v)
    plsc.subcore_barrier()

    base_w = w * _EPW

    def pair(j, carry):
        a = 2 * j
        da = pltpu.async_copy(h_hbm.at[ridx_v.at[a]], rows_a, gsem_a)
        dca = pltpu.async_copy(
            colp_hbm.at[pl.ds(base_w + a * _CH, _CH)], cidx_a, csem_a)
        db = pltpu.async_copy(h_hbm.at[ridx_v.at[a + 1]], rows_b, gsem_b)
        dcb = pltpu.async_copy(
            colp_hbm.at[pl.ds(base_w + (a + 1) * _CH, _CH)], cidx_b, csem_b)
        da.wait()
        dca.wait()
        pltpu.sync_copy(rows_a, s_sh.at[cidx_a], add=True)
        db.wait()
        dcb.wait()
        pltpu.sync_copy(rows_b, s_sh.at[cidx_b], add=True)
        return carry

    lax.fori_loop(0, _NCH // 2, pair, 0)
    plsc.subcore_barrier()

    def _out(off, sz):
        _per_core_out(
            c, lambda o: pltpu.sync_copy(s_sh.at[pl.ds(off, sz)],
                                         o.at[pl.ds(off, sz)]),
            sa_out, sb_out)

    _per_subcore_slices(s, _out)


def _sc_agg_body(ea_hbm, ridx3_hbm, z128,
                 agga_out, aggb_out,
                 ridx_v, ea_a, ea_b, lsem_a, lsem_b, ssem_a, ssem_b, agg_sh):
    c = lax.axis_index("c")
    s = lax.axis_index("s")
    _per_subcore_slices(
        s, lambda off, sz: pltpu.sync_copy(z128.at[pl.ds(off, sz)],
                                           agg_sh.at[pl.ds(off, sz)]))
    w = c * _NS + s
    pltpu.sync_copy(ridx3_hbm.at[w], ridx_v)
    plsc.subcore_barrier()

    base_w = w * _EPW

    def pair(j, carry):
        a = 2 * j
        da = pltpu.async_copy(
            ea_hbm.at[pl.ds(base_w + a * _CH, _CH)], ea_a, lsem_a)
        db = pltpu.async_copy(
            ea_hbm.at[pl.ds(base_w + (a + 1) * _CH, _CH)], ea_b, lsem_b)
        da.wait()
        pltpu.sync_copy(ea_a, agg_sh.at[ridx_v.at[a]], add=True)
        db.wait()
        pltpu.sync_copy(ea_b, agg_sh.at[ridx_v.at[a + 1]], add=True)
        return carry

    lax.fori_loop(0, _NCH // 2, pair, 0)
    plsc.subcore_barrier()

    def _out(off, sz):
        _per_core_out(
            c, lambda o: pltpu.sync_copy(agg_sh.at[pl.ds(off, sz)],
                                         o.at[pl.ds(off, sz)]),
            agga_out, aggb_out)

    _per_subcore_slices(s, _out)


_f32 = jnp.float32
_nd = jax.ShapeDtypeStruct((_N, _D), _f32)
_nh = jax.ShapeDtypeStruct((_N, _HD), _f32)
_na = jax.ShapeDtypeStruct((_N, _EA), _f32)


@functools.cache
def _build_sc_kernels():
    mesh = plsc.VectorSubcoreMesh(core_axis_name="c", subcore_axis_name="s")
    sc_spmm = pl.kernel(
        _sc_spmm_body,
        out_type=(_nd, _nd),
        mesh=mesh,
        scratch_types=[
            pltpu.VMEM((_NCH, _CH), jnp.int32),
            pltpu.VMEM((_CH,), jnp.int32),
            pltpu.VMEM((_CH,), jnp.int32),
            pltpu.VMEM((_CH, _D), jnp.float32),
            pltpu.VMEM((_CH, _D), jnp.float32),
            pltpu.SemaphoreType.DMA,
            pltpu.SemaphoreType.DMA,
            pltpu.SemaphoreType.DMA,
            pltpu.SemaphoreType.DMA,
            pltpu.SemaphoreType.DMA,
            pltpu.SemaphoreType.DMA,
            pltpu.VMEM_SHARED((_N + _TR, _D), jnp.float32),
        ],
    )
    sc_agg = pl.kernel(
        _sc_agg_body,
        out_type=(_na, _na),
        mesh=mesh,
        scratch_types=[
            pltpu.VMEM((_NCH, _CH), jnp.int32),
            pltpu.VMEM((_CH, _EA), jnp.float32),
            pltpu.VMEM((_CH, _EA), jnp.float32),
            pltpu.SemaphoreType.DMA,
            pltpu.SemaphoreType.DMA,
            pltpu.SemaphoreType.DMA,
            pltpu.SemaphoreType.DMA,
            pltpu.VMEM_SHARED((_N, _EA), jnp.float32),
        ],
    )
    return sc_spmm, sc_agg


def _bn(y, g, b, eps=1e-5):
    m = jnp.mean(y, axis=0, keepdims=True)
    v = jnp.mean((y - m) * (y - m), axis=0, keepdims=True)
    return g * (y - m) * lax.rsqrt(v + eps) + b


def _dot(a, w):
    return jnp.dot(a, w, preferred_element_type=jnp.float32)


def _tc_pre_body(x_ref, Wnb_ref, bnb_ref, Wn_ref, bn_ref, gn_ref, btn_ref,
                 h_ref, xn_ref):
    x = x_ref[...]
    h_ref[...] = _dot(x, Wnb_ref[...]) + bnb_ref[...]
    xn_ref[...] = _bn(_dot(x, Wn_ref[...]) + bn_ref[...], gn_ref[...], btn_ref[...])


def _tc_layer0_body(sa_ref, sb_ref, agga_ref, aggb_ref, xn0_ref,
                    Wea_ref, Wm1_ref, bm1_ref, Wm2_ref, bm2_ref,
                    ge_ref, bte_ref, gnb_ref, btnb_ref,
                    gm1_ref, btm1_ref, gm2_ref, btm2_ref,
                    ea_ref, x1_ref):
    msg = sa_ref[...] + sb_ref[...]
    agg = agga_ref[...] + aggb_ref[...]
    ea_agg = _bn(_dot(agg, Wea_ref[...]), ge_ref[...], bte_ref[...])
    ea_ref[...] = ea_agg
    out = jnp.maximum(
        xn0_ref[...] + _bn(msg, gnb_ref[...], btnb_ref[...]) + ea_agg, 0.0)
    out = _bn(_dot(out, Wm1_ref[...]) + bm1_ref[...], gm1_ref[...], btm1_ref[...])
    out = jnp.maximum(out, 0.0)
    out = _bn(_dot(out, Wm2_ref[...]) + bm2_ref[...], gm2_ref[...], btm2_ref[...])
    x1_ref[...] = jnp.maximum(out, 0.0)


def _tc_layer1_body(sa_ref, sb_ref, xn1_ref, ea_ref,
                    Wm1_ref, bm1_ref, Wm2_ref, bm2_ref,
                    gnb_ref, btnb_ref, gm1_ref, btm1_ref, gm2_ref, btm2_ref,
                    x2_ref):
    msg = sa_ref[...] + sb_ref[...]
    out = jnp.maximum(
        xn1_ref[...] + _bn(msg, gnb_ref[...], btnb_ref[...]) + ea_ref[...], 0.0)
    out = _bn(_dot(out, Wm1_ref[...]) + bm1_ref[...], gm1_ref[...], btm1_ref[...])
    out = jnp.maximum(out, 0.0)
    out = _bn(_dot(out, Wm2_ref[...]) + bm2_ref[...], gm2_ref[...], btm2_ref[...])
    x2_ref[...] = jnp.maximum(out, 0.0)


_tc_pre = pl.pallas_call(_tc_pre_body, out_shape=(_nd, _nd))
_tc_layer0 = pl.pallas_call(_tc_layer0_body, out_shape=(_nd, _nd))
_tc_layer1 = pl.pallas_call(_tc_layer1_body, out_shape=_nd)


def kernel(node_attr, edge_index, edge_attr,
           W_node0, b_node0, W_node1, b_node1,
           W_nb0, b_nb0, W_nb1, b_nb1,
           W_edge, b_edge, W_m1, b_m1, W_m2, b_m2,
           g_bn_node, beta_bn_node, g_bn_edge, beta_bn_edge,
           g_bn_nb, beta_bn_nb, g_bn_m1, beta_bn_m1,
           g_bn_m2, beta_bn_m2):
    pad = _EPAD - _E
    rowp = jnp.concatenate([edge_index[0], jnp.arange(pad, dtype=jnp.int32) % _N])
    colp = jnp.concatenate(
        [edge_index[1], _N + (jnp.arange(pad, dtype=jnp.int32) % _TR)])
    row3 = rowp.reshape(_NW, _NCH, _CH)
    ea_aug = jnp.concatenate(
        [jnp.concatenate([edge_attr, jnp.ones((_E, 1), _f32),
                          jnp.zeros((_E, _EA - 17), _f32)], axis=1),
         jnp.zeros((pad, _EA), _f32)], axis=0)
    W_ea = jnp.concatenate(
        [W_edge, b_edge[None, :], jnp.zeros((_EA - 17, _D), _f32)], axis=0)
    z128 = jnp.zeros((_N, _EA), _f32)
    r = lambda v: v[None, :]

    h0, xn0 = _tc_pre(node_attr, W_nb0, r(b_nb0), W_node0, r(b_node0),
                      r(g_bn_node), r(beta_bn_node))

    sc_spmm, sc_agg = _build_sc_kernels()
    s0a, s0b = sc_spmm(h0, row3, colp, z128)
    z128_dep = z128 + s0a[:, :1] * 0.0
    agga, aggb = sc_agg(ea_aug, row3, z128_dep)

    ea_agg, x1 = _tc_layer0(
        s0a, s0b, agga, aggb, xn0,
        W_ea, W_m1, r(b_m1), W_m2, r(b_m2),
        r(g_bn_edge), r(beta_bn_edge),
        r(g_bn_nb), r(beta_bn_nb), r(g_bn_m1), r(beta_bn_m1),
        r(g_bn_m2), r(beta_bn_m2))

    h1, xn1 = _tc_pre(x1, W_nb1, r(b_nb1), W_node1, r(b_node1),
                      r(g_bn_node), r(beta_bn_node))

    s1a, s1b = sc_spmm(h1, row3, colp, z128)

    x2 = _tc_layer1(
        s1a, s1b, xn1, ea_agg,
        W_m1, r(b_m1), W_m2, r(b_m2),
        r(g_bn_nb), r(beta_bn_nb), r(g_bn_m1), r(beta_bn_m1),
        r(g_bn_m2), r(beta_bn_m2))
    return x2
